# baseline (speedup 1.0000x reference)
import jax
import jax.numpy as jnp
from jax.experimental import pallas as pl
from jax.experimental.pallas import tpu as pltpu

C = 4096
RANK = 2048
_XG = 32
_WG = 64
_RB = 256
_PB = 256


def _fold8(acc):
    b = acc[0:4, :] + acc[4:8, :]
    c2 = b[0:2, :] + b[2:4, :]
    return c2[0:1, :] + c2[1:2, :]


def _ssq_body(x_ref, o_ref, acc_ref):
    i = pl.program_id(0)

    @pl.when(i == 0)
    def _():
        acc_ref[...] = jnp.zeros_like(acc_ref)

    xb = x_ref[...]
    acc = acc_ref[...]
    for g in range(_XG):
        for n in range(4):
            sl = xb[n, g * 8:(g + 1) * 8, :]
            acc = acc + sl * sl
    acc_ref[...] = acc

    @pl.when(i == pl.num_programs(0) - 1)
    def _():
        o_ref[...] = jnp.sqrt(_fold8(acc_ref[...]))


def _metric_body(w_ref, xn_ref, o_ref, acc_ref):
    i = pl.program_id(0)

    @pl.when(i == 0)
    def _():
        acc_ref[...] = jnp.zeros_like(acc_ref)

    wb = w_ref[...]
    xn = xn_ref[...]
    acc = acc_ref[...]
    for g in range(_WG):
        acc = acc + jnp.abs(wb[g * 8:(g + 1) * 8, :]) * xn
    acc_ref[...] = acc

    @pl.when(i == pl.num_programs(0) - 1)
    def _():
        o_ref[...] = _fold8(acc_ref[...])


def _srl31(x):
    return jax.lax.shift_right_logical(x, 31)


def _sort_body(mrow_ref, o_ref):
    k = jax.lax.bitcast_convert_type(mrow_ref[...], jnp.int32)
    lane = jax.lax.broadcasted_iota(jnp.int32, (32, 128), 1)
    row = jax.lax.broadcasted_iota(jnp.int32, (32, 128), 0)
    v = row * 128 + lane
    for kst_log in range(1, 13):
        kst = 1 << kst_log
        for j_log in range(kst_log - 1, -1, -1):
            j = 1 << j_log
            if j < 128:
                u = (lane & j) != 0
                pk = jnp.where(u, pltpu.roll(k, j, axis=1),
                               pltpu.roll(k, 128 - j, axis=1))
                pv = jnp.where(u, pltpu.roll(v, j, axis=1),
                               pltpu.roll(v, 128 - j, axis=1))
                less = (pk < k) | ((pk == k) & (pv < v))
                if kst >= 128:
                    dsc = (row & (kst >> 7)) != 0
                else:
                    dsc = (lane & kst) != 0
                take = less ^ u ^ dsc
                k = jnp.where(take, pk, k)
                v = jnp.where(take, pv, v)
            else:
                jr = j >> 7
                nk, nv = [], []
                for base in range(0, 32, 2 * jr):
                    ak = k[base:base + jr]
                    bk = k[base + jr:base + 2 * jr]
                    av = v[base:base + jr]
                    bv = v[base + jr:base + 2 * jr]
                    less = (bk < ak) | ((bk == ak) & (bv < av))
                    if (base & (kst >> 7)) == 0:
                        nk += [jnp.where(less, bk, ak), jnp.where(less, ak, bk)]
                        nv += [jnp.where(less, bv, av), jnp.where(less, av, bv)]
                    else:
                        nk += [jnp.where(less, ak, bk), jnp.where(less, bk, ak)]
                        nv += [jnp.where(less, av, bv), jnp.where(less, bv, av)]
                k = jnp.concatenate(nk, axis=0)
                v = jnp.concatenate(nv, axis=0)
    o_ref[...] = v[0:16, :]


def kernel(W, X):
    n, l, c = X.shape

    xn = pl.pallas_call(
        _ssq_body,
        grid=(l // (8 * _XG),),
        in_specs=[pl.BlockSpec((n, 8 * _XG, c), lambda i: (0, i, 0))],
        out_specs=pl.BlockSpec((1, c), lambda i: (0, 0)),
        out_shape=jax.ShapeDtypeStruct((1, c), jnp.float32),
        scratch_shapes=[pltpu.VMEM((8, c), jnp.float32)],
    )(X)

    metric = pl.pallas_call(
        _metric_body,
        grid=(W.shape[0] // (8 * _WG),),
        in_specs=[
            pl.BlockSpec((8 * _WG, c), lambda i: (i, 0)),
            pl.BlockSpec((1, c), lambda i: (0, 0)),
        ],
        out_specs=pl.BlockSpec((1, c), lambda i: (0, 0)),
        out_shape=jax.ShapeDtypeStruct((1, c), jnp.float32),
        scratch_shapes=[pltpu.VMEM((8, c), jnp.float32)],
    )(W, xn)

    out = pl.pallas_call(
        _sort_body,
        grid=(1,),
        in_specs=[pl.BlockSpec((32, 128), lambda i: (0, 0))],
        out_specs=pl.BlockSpec((16, 128), lambda i: (0, 0)),
        out_shape=jax.ShapeDtypeStruct((16, 128), jnp.int32),
    )(metric.reshape(32, 128))

    return out.reshape(RANK)


# speedup vs baseline: 1.0545x; 1.0545x over previous
import jax
import jax.numpy as jnp
from jax.experimental import pallas as pl
from jax.experimental.pallas import tpu as pltpu

C = 4096
RANK = 2048
_XG = 16
_WG = 64
_RB = 256
_PB = 256


def _fold8(acc):
    b = acc[0:4, :] + acc[4:8, :]
    c2 = b[0:2, :] + b[2:4, :]
    return c2[0:1, :] + c2[1:2, :]


def _ssq_body(x_ref, o_ref, acc_ref):
    i = pl.program_id(0)

    @pl.when(i == 0)
    def _():
        acc_ref[...] = jnp.zeros_like(acc_ref)

    xb = x_ref[...]
    acc = acc_ref[...]
    for g in range(_XG):
        for n in range(4):
            sl = xb[n, g * 8:(g + 1) * 8, :]
            acc = acc + sl * sl
    acc_ref[...] = acc

    @pl.when(i == pl.num_programs(0) - 1)
    def _():
        o_ref[...] = jnp.sqrt(_fold8(acc_ref[...]))


def _metric_body(w_ref, xn_ref, o_ref, idx_ref, acc_ref):
    i = pl.program_id(0)

    @pl.when(i == 0)
    def _():
        acc_ref[...] = jnp.zeros_like(acc_ref)

    wb = w_ref[...]
    xn = xn_ref[...]
    acc = acc_ref[...]
    for g in range(_WG):
        acc = acc + jnp.abs(wb[g * 8:(g + 1) * 8, :]) * xn
    acc_ref[...] = acc

    @pl.when(i == pl.num_programs(0) - 1)
    def _():
        met = _fold8(acc_ref[...])
        o_ref[...] = met
        _bitonic(met.reshape(32, 128), idx_ref)


def _srl31(x):
    return jax.lax.shift_right_logical(x, 31)


def _bitonic(m32, o_ref):
    k = jax.lax.bitcast_convert_type(m32, jnp.int32)
    lane = jax.lax.broadcasted_iota(jnp.int32, (32, 128), 1)
    row = jax.lax.broadcasted_iota(jnp.int32, (32, 128), 0)
    v = row * 128 + lane
    for kst_log in range(1, 13):
        kst = 1 << kst_log
        for j_log in range(kst_log - 1, -1, -1):
            j = 1 << j_log
            if j < 128:
                u = (lane & j) != 0
                pk = jnp.where(u, pltpu.roll(k, j, axis=1),
                               pltpu.roll(k, 128 - j, axis=1))
                pv = jnp.where(u, pltpu.roll(v, j, axis=1),
                               pltpu.roll(v, 128 - j, axis=1))
                less = (pk < k) | ((pk == k) & (pv < v))
                if kst >= 128:
                    dsc = (row & (kst >> 7)) != 0
                else:
                    dsc = (lane & kst) != 0
                take = less ^ u ^ dsc
                k = jnp.where(take, pk, k)
                v = jnp.where(take, pv, v)
            else:
                jr = j >> 7
                nk, nv = [], []
                for base in range(0, 32, 2 * jr):
                    ak = k[base:base + jr]
                    bk = k[base + jr:base + 2 * jr]
                    av = v[base:base + jr]
                    bv = v[base + jr:base + 2 * jr]
                    less = (bk < ak) | ((bk == ak) & (bv < av))
                    if (base & (kst >> 7)) == 0:
                        nk += [jnp.where(less, bk, ak), jnp.where(less, ak, bk)]
                        nv += [jnp.where(less, bv, av), jnp.where(less, av, bv)]
                    else:
                        nk += [jnp.where(less, ak, bk), jnp.where(less, bk, ak)]
                        nv += [jnp.where(less, av, bv), jnp.where(less, bv, av)]
                k = jnp.concatenate(nk, axis=0)
                v = jnp.concatenate(nv, axis=0)
    o_ref[...] = v[0:16, :]


def kernel(W, X):
    n, l, c = X.shape

    xn = pl.pallas_call(
        _ssq_body,
        grid=(l // (8 * _XG),),
        in_specs=[pl.BlockSpec((n, 8 * _XG, c), lambda i: (0, i, 0))],
        out_specs=pl.BlockSpec((1, c), lambda i: (0, 0)),
        out_shape=jax.ShapeDtypeStruct((1, c), jnp.float32),
        scratch_shapes=[pltpu.VMEM((8, c), jnp.float32)],
    )(X)

    metric = pl.pallas_call(
        _metric_body,
        grid=(W.shape[0] // (8 * _WG),),
        in_specs=[
            pl.BlockSpec((8 * _WG, c), lambda i: (i, 0)),
            pl.BlockSpec((1, c), lambda i: (0, 0)),
        ],
        out_specs=[pl.BlockSpec((1, c), lambda i: (0, 0)),
                   pl.BlockSpec((16, 128), lambda i: (0, 0))],
        out_shape=[jax.ShapeDtypeStruct((1, c), jnp.float32),
                   jax.ShapeDtypeStruct((16, 128), jnp.int32)],
        scratch_shapes=[pltpu.VMEM((8, c), jnp.float32)],
    )(W, xn)

    metric, out = metric

    return out.reshape(RANK)
